# baseline (speedup 1.0000x reference)
import functools

import jax
import jax.numpy as jnp
from jax import lax
from jax.experimental import pallas as pl
from jax.experimental.pallas import tpu as pltpu
from jax.experimental.pallas import tpu_sc as plsc

NE = 8192
D = 256
TM = 256
NT = NE // TM
NTOK = 8192
FM = 1024
BIG = 2 ** 30


KB = 2048
KT = NE // KB


def _argmin_body(z2_ref, w2_ref, zb2_ref, wbt_ref, idx_ref):
    zb2 = zb2_ref[...]
    z2 = z2_ref[...]

    m = None
    vj = None
    for t in range(KT):
        wbt_t = wbt_ref[:, t * KB:(t + 1) * KB]
        mm2 = lax.dot_general(zb2, wbt_t, (((1,), (0,)), ((), ())),
                              preferred_element_type=jnp.float32)
        s = (z2 + w2_ref[:, t * KB:(t + 1) * KB]) - mm2
        if m is None:
            m = s
            vj = jnp.zeros(s.shape, jnp.int32)
        else:
            upd = s < m
            m = jnp.where(upd, s, m)
            vj = jnp.where(upd, t, vj)

    gm = jnp.min(m, axis=1, keepdims=True)
    ks = vj * KB + lax.broadcasted_iota(jnp.int32, m.shape, 1)
    key = jnp.where(m == gm, ks, BIG)
    idx_ref[0, 0, :] = jnp.min(key, axis=1)


_argmin_call = pl.pallas_call(
    _argmin_body,
    grid=(NT,),
    in_specs=[
        pl.BlockSpec((TM, 1), lambda i: (i, 0)),
        pl.BlockSpec((1, NE), lambda i: (0, 0)),
        pl.BlockSpec((TM, D), lambda i: (i, 0)),
        pl.BlockSpec((D, NE), lambda i: (0, 0)),
    ],
    out_specs=pl.BlockSpec((1, 1, TM), lambda i: (i, 0, 0)),
    out_shape=jax.ShapeDtypeStruct((NT, 1, TM), jnp.int32),
)


_NC = 2
_NS = 16
_NW = _NC * _NS
_BPW = NTOK // _NW


@functools.cache
def _make_sc_gather():
    @functools.partial(
        pl.kernel,
        mesh=plsc.VectorSubcoreMesh(core_axis_name="c", subcore_axis_name="s",
                                    num_cores=_NC, num_subcores=_NS),
        out_type=jax.ShapeDtypeStruct((NTOK, D), jnp.float32),
        scratch_types=[
            pltpu.VMEM((_BPW,), jnp.int32),
            pltpu.VMEM((_BPW, D), jnp.float32),
            pltpu.SemaphoreType.DMA,
        ],
    )
    def _sc_gather(table_hbm, idx_hbm, out_hbm, idx_v, rows_v, sem):
        wid = lax.axis_index("s") * _NC + lax.axis_index("c")
        base = wid * _BPW
        pltpu.sync_copy(idx_hbm.at[pl.ds(base, _BPW)], idx_v)
        pltpu.async_copy(table_hbm.at[idx_v], rows_v, sem).wait()
        pltpu.sync_copy(rows_v, out_hbm.at[pl.ds(base, _BPW)])

    return _sc_gather


def _finish_body(z_ref, q_ref, qst_ref, loss_ref):
    i = pl.program_id(0)
    zv = z_ref[...]
    qv = q_ref[...]
    d = qv - zv
    qst_ref[...] = zv + d
    part = jnp.reshape(jnp.sum(d * d), (1, 1))

    @pl.when(i == 0)
    def _init():
        loss_ref[...] = part

    @pl.when(i != 0)
    def _acc():
        loss_ref[...] = loss_ref[...] + part


_finish_call = pl.pallas_call(
    _finish_body,
    grid=(NTOK // FM,),
    in_specs=[
        pl.BlockSpec((FM, D), lambda i: (i, 0)),
        pl.BlockSpec((FM, D), lambda i: (i, 0)),
    ],
    out_specs=[
        pl.BlockSpec((FM, D), lambda i: (i, 0)),
        pl.BlockSpec((1, 1), lambda i: (0, 0)),
    ],
    out_shape=[
        jax.ShapeDtypeStruct((NTOK, D), jnp.float32),
        jax.ShapeDtypeStruct((1, 1), jnp.float32),
    ],
)


def kernel(z, W):
    input_shape = z.shape
    flat = z.reshape(-1, D)
    z2 = jnp.sum(z ** 2, axis=3).reshape(-1, 1)
    w2 = jnp.sum(W ** 2, axis=1).reshape(1, NE)
    zb2 = (jnp.float32(2.0) * flat).astype(jnp.bfloat16)
    wbt = W.astype(jnp.bfloat16).T

    idx = _argmin_call(z2, w2, zb2, wbt).reshape(NTOK)
    quantized = _make_sc_gather()(W, idx)
    qst, loss_sum = _finish_call(flat, quantized)

    mean_sq = loss_sum[0, 0] / jnp.float32(NTOK * D)
    vq_loss = mean_sq + jnp.float32(0.25) * mean_sq
    return (qst.reshape(input_shape), vq_loss, idx.reshape(input_shape[:-1]))


# speedup vs baseline: 1.0466x; 1.0466x over previous
import functools

import jax
import jax.numpy as jnp
from jax import lax
from jax.experimental import pallas as pl
from jax.experimental.pallas import tpu as pltpu
from jax.experimental.pallas import tpu_sc as plsc

NE = 8192
D = 256
TM = 512
NT = NE // TM
NTOK = 8192
FM = 1024
BIG = 2 ** 30


KB = 1024
KT = NE // KB


def _argmin_body(z2_ref, w2_ref, zb2_ref, wbt_ref, idx_ref):
    zb2 = zb2_ref[...]
    z2 = z2_ref[...]

    m = None
    vj = None
    for t in range(KT):
        wbt_t = wbt_ref[:, t * KB:(t + 1) * KB]
        mm2 = lax.dot_general(zb2, wbt_t, (((1,), (0,)), ((), ())),
                              preferred_element_type=jnp.float32)
        s = (z2 + w2_ref[:, t * KB:(t + 1) * KB]) - mm2
        if m is None:
            m = s
            vj = jnp.zeros(s.shape, jnp.int32)
        else:
            upd = s < m
            m = jnp.where(upd, s, m)
            vj = jnp.where(upd, t, vj)

    gm = jnp.min(m, axis=1, keepdims=True)
    ks = vj * KB + lax.broadcasted_iota(jnp.int32, m.shape, 1)
    key = jnp.where(m == gm, ks, BIG)
    idx_ref[0, 0, :] = jnp.min(key, axis=1)


_argmin_call = pl.pallas_call(
    _argmin_body,
    grid=(NT,),
    in_specs=[
        pl.BlockSpec((TM, 1), lambda i: (i, 0)),
        pl.BlockSpec((1, NE), lambda i: (0, 0)),
        pl.BlockSpec((TM, D), lambda i: (i, 0)),
        pl.BlockSpec((D, NE), lambda i: (0, 0)),
    ],
    out_specs=pl.BlockSpec((1, 1, TM), lambda i: (i, 0, 0)),
    out_shape=jax.ShapeDtypeStruct((NT, 1, TM), jnp.int32),
)


_NC = 2
_NS = 16
_NW = _NC * _NS
_BPW = NTOK // _NW


@functools.cache
def _make_sc_gather():
    @functools.partial(
        pl.kernel,
        mesh=plsc.VectorSubcoreMesh(core_axis_name="c", subcore_axis_name="s",
                                    num_cores=_NC, num_subcores=_NS),
        out_type=jax.ShapeDtypeStruct((NTOK, D), jnp.float32),
        scratch_types=[
            pltpu.VMEM((_BPW,), jnp.int32),
            pltpu.VMEM((_BPW, D), jnp.float32),
            pltpu.SemaphoreType.DMA,
        ],
    )
    def _sc_gather(table_hbm, idx_hbm, out_hbm, idx_v, rows_v, sem):
        wid = lax.axis_index("s") * _NC + lax.axis_index("c")
        base = wid * _BPW
        pltpu.sync_copy(idx_hbm.at[pl.ds(base, _BPW)], idx_v)
        pltpu.async_copy(table_hbm.at[idx_v], rows_v, sem).wait()
        pltpu.sync_copy(rows_v, out_hbm.at[pl.ds(base, _BPW)])

    return _sc_gather


def _finish_body(z_ref, q_ref, qst_ref, loss_ref):
    i = pl.program_id(0)
    zv = z_ref[...]
    qv = q_ref[...]
    d = qv - zv
    qst_ref[...] = zv + d
    part = jnp.reshape(jnp.sum(d * d), (1, 1))

    @pl.when(i == 0)
    def _init():
        loss_ref[...] = part

    @pl.when(i != 0)
    def _acc():
        loss_ref[...] = loss_ref[...] + part


_finish_call = pl.pallas_call(
    _finish_body,
    grid=(NTOK // FM,),
    in_specs=[
        pl.BlockSpec((FM, D), lambda i: (i, 0)),
        pl.BlockSpec((FM, D), lambda i: (i, 0)),
    ],
    out_specs=[
        pl.BlockSpec((FM, D), lambda i: (i, 0)),
        pl.BlockSpec((1, 1), lambda i: (0, 0)),
    ],
    out_shape=[
        jax.ShapeDtypeStruct((NTOK, D), jnp.float32),
        jax.ShapeDtypeStruct((1, 1), jnp.float32),
    ],
)


def kernel(z, W):
    input_shape = z.shape
    flat = z.reshape(-1, D)
    z2 = jnp.sum(z ** 2, axis=3).reshape(-1, 1)
    w2 = jnp.sum(W ** 2, axis=1).reshape(1, NE)
    zb2 = (jnp.float32(2.0) * flat).astype(jnp.bfloat16)
    wbt = W.astype(jnp.bfloat16).T

    idx = _argmin_call(z2, w2, zb2, wbt).reshape(NTOK)
    quantized = _make_sc_gather()(W, idx)
    qst, loss_sum = _finish_call(flat, quantized)

    mean_sq = loss_sum[0, 0] / jnp.float32(NTOK * D)
    vq_loss = mean_sq + jnp.float32(0.25) * mean_sq
    return (qst.reshape(input_shape), vq_loss, idx.reshape(input_shape[:-1]))
